# Initial kernel scaffold; baseline (speedup 1.0000x reference)
import jax
import jax.numpy as jnp
from jax.experimental import pallas as pl


def kernel(x, mask, edge_index, W1, b1, logp, means, logvars, W_mu, b_mu, W_lv, b_lv):
    raise NotImplementedError("write your pallas kernel here")


# speedup vs baseline: 7.6551x; 7.6551x over previous
import functools

import jax
import jax.numpy as jnp
from jax.experimental import pallas as pl
from jax.experimental.pallas import tpu as pltpu

N = 10000
D_FEAT = 128
NHID = 32
LATENT = 16
K = 5

BM = 512
BN = 1024


def _decoder_body(z_row_ref, z_col_ref, out_ref):
    zi = z_row_ref[...]
    zj = z_col_ref[...]
    logits = jax.lax.dot_general(
        zi, zj, (((1,), (1,)), ((), ())), preferred_element_type=jnp.float32
    )
    out_ref[...] = jax.nn.sigmoid(logits)


def _decoder(z):
    n = z.shape[0]
    grid = (pl.cdiv(n, BM), pl.cdiv(n, BN))
    return pl.pallas_call(
        _decoder_body,
        grid=grid,
        in_specs=[
            pl.BlockSpec((BM, LATENT), lambda i, j: (i, 0)),
            pl.BlockSpec((BN, LATENT), lambda i, j: (j, 0)),
        ],
        out_specs=pl.BlockSpec((BM, BN), lambda i, j: (i, j)),
        out_shape=jax.ShapeDtypeStruct((n, n), jnp.float32),
    )(z, z)


def _spmm(vals, rows, cols, X, n):
    return jax.ops.segment_sum(vals[:, None] * X[cols], rows, num_segments=n)


def kernel(x, mask, edge_index, W1, b1, logp, means, logvars, W_mu, b_mu, W_lv, b_lv):
    n = x.shape[0]
    src = edge_index[0]
    dst = edge_index[1]
    loop = jnp.arange(n)
    rows = jnp.concatenate([dst, loop])
    cols = jnp.concatenate([src, loop])
    ones = jnp.ones(rows.shape[0], jnp.float32)
    deg_r = jax.ops.segment_sum(ones, rows, num_segments=n)
    deg_c = jax.ops.segment_sum(ones, cols, num_segments=n)
    vals = ones / jnp.sqrt(deg_r[rows] * deg_c[cols])
    vals2 = vals * vals

    d_in = x.shape[1]
    variances = jnp.exp(logvars)
    Mf = mask.astype(jnp.float32)
    xm = jnp.where(mask, 0.0, x)

    TX0 = xm @ W1
    G1 = jnp.concatenate([TX0, Mf], axis=1)
    S1 = _spmm(vals, rows, cols, G1, n)
    S2 = _spmm(vals2, rows, cols, Mf, n)
    S0 = S1[:, :NHID]
    SM1 = S1[:, NHID:]
    SM2 = S2

    Bk = means[:, :, None] * W1[None, :, :]
    Ck = variances[:, :, None] * (W1 * W1)[None, :, :]
    cx = S0[None] + jnp.einsum('nd,kdh->knh', SM1, Bk) + b1
    cc = jnp.einsum('nd,kdh->knh', SM2, Ck)

    std = jnp.sqrt(cc + 1e-8)
    r = cx / std
    cdf = 0.5 * (1.0 + jax.lax.erf(r / jnp.sqrt(2.0)))
    pdf = jnp.exp(-0.5 * r * r) / jnp.sqrt(2.0 * jnp.pi)
    expected_relu = cx * cdf + std * pdf

    U = 1.0 - Mf
    xm2 = xm * xm
    V1 = (1.0 / variances).T
    V2 = (means / variances).T
    V3 = (means * means / variances).T
    quad = xm2 @ V1 - 2.0 * (xm @ V2) + U @ V3
    const_k = 0.5 * d_in * jnp.log(2.0 * jnp.pi) + 0.5 * jnp.sum(logvars, axis=1)
    log_n = (-0.5 * quad - const_k[None, :]).T
    gamma = jax.nn.softmax(logp[:, None] + log_n, axis=0)
    h1 = jnp.sum(gamma[:, :, None] * expected_relu, axis=0)

    H2 = h1 @ jnp.concatenate([W_mu, W_lv], axis=1)
    S3 = _spmm(vals, rows, cols, H2, n)
    mu = S3[:, :LATENT] + b_mu
    logvar = S3[:, LATENT:] + b_lv
    z = mu
    adj_recon = _decoder(z)
    return (adj_recon, z, mu, logvar)


# speedup vs baseline: 58.3898x; 7.6276x over previous
import functools

import jax
import jax.numpy as jnp
from jax import lax
from jax.experimental import pallas as pl
from jax.experimental.pallas import tpu as pltpu
from jax.experimental.pallas import tpu_sc as plsc

N = 10000
E = 320000
D_FEAT = 128
NHID = 32
LATENT = 16
K = 5

NP = 10240
RPT = NP // 16
C = 128
EPAD = 323584
EPT1 = EPAD // 16
EPT2 = EPAD // 32
W1COL = NHID + D_FEAT

_MESH = plsc.VectorSubcoreMesh(
    core_axis_name="c", subcore_axis_name="s", num_cores=2, num_subcores=16
)


@functools.partial(
    pl.kernel,
    out_type=jax.ShapeDtypeStruct((2 * NP, 8), jnp.float32),
    mesh=_MESH,
    compiler_params=pltpu.CompilerParams(use_tc_tiling_on_sc=False),
    scratch_types=[
        pltpu.VMEM((C,), jnp.int32),
        pltpu.VMEM((C, 8), jnp.float32),
        pltpu.VMEM_SHARED((NP, 8), jnp.float32),
    ],
)
def _sc_degree(idx2_hbm, ones_hbm, zeros_hbm, out_hbm, idx_v, ones_v, acc):
    c = lax.axis_index("c")
    s = lax.axis_index("s")
    pltpu.sync_copy(zeros_hbm, acc.at[pl.ds(s * RPT, RPT)])
    pltpu.sync_copy(ones_hbm, ones_v)
    plsc.subcore_barrier()

    def body(i, carry):
        base = c * EPAD + s * EPT1 + i * C
        pltpu.sync_copy(idx2_hbm.at[pl.ds(base, C)], idx_v)
        pltpu.sync_copy(ones_v, acc.at[idx_v], add=True)
        return carry

    lax.fori_loop(0, EPT1 // C, body, 0)
    plsc.subcore_barrier()
    pltpu.sync_copy(
        acc.at[pl.ds(s * RPT, RPT)], out_hbm.at[pl.ds(c * NP + s * RPT, RPT)]
    )


@functools.partial(
    pl.kernel,
    out_type=jax.ShapeDtypeStruct((2 * NP, W1COL), jnp.float32),
    mesh=_MESH,
    compiler_params=pltpu.CompilerParams(use_tc_tiling_on_sc=False),
    scratch_types=[
        pltpu.VMEM((C,), jnp.int32),
        pltpu.VMEM((C,), jnp.int32),
        pltpu.VMEM((C, W1COL), jnp.float32),
        pltpu.VMEM_SHARED((NP, W1COL), jnp.float32),
        pltpu.SemaphoreType.DMA,
    ],
)
def _sc_spmm1(gbig_hbm, src2_hbm, dst_hbm, zeros_hbm, out_hbm,
              sidx_v, didx_v, rows_v, acc, sem):
    c = lax.axis_index("c")
    s = lax.axis_index("s")
    pltpu.sync_copy(zeros_hbm, acc.at[pl.ds(s * RPT, RPT)])
    plsc.subcore_barrier()

    def body(i, carry):
        base = s * EPT1 + i * C
        pltpu.sync_copy(src2_hbm.at[pl.ds(c * EPAD + base, C)], sidx_v)
        pltpu.async_copy(gbig_hbm.at[sidx_v], rows_v, sem).wait()
        pltpu.sync_copy(dst_hbm.at[pl.ds(base, C)], didx_v)
        pltpu.sync_copy(rows_v, acc.at[didx_v], add=True)
        return carry

    lax.fori_loop(0, EPT1 // C, body, 0)
    plsc.subcore_barrier()
    pltpu.sync_copy(
        acc.at[pl.ds(s * RPT, RPT)], out_hbm.at[pl.ds(c * NP + s * RPT, RPT)]
    )


@functools.partial(
    pl.kernel,
    out_type=jax.ShapeDtypeStruct((2 * NP, NHID), jnp.float32),
    mesh=_MESH,
    compiler_params=pltpu.CompilerParams(use_tc_tiling_on_sc=False),
    scratch_types=[
        pltpu.VMEM((C,), jnp.int32),
        pltpu.VMEM((C,), jnp.int32),
        pltpu.VMEM((C, NHID), jnp.float32),
        pltpu.VMEM_SHARED((NP, NHID), jnp.float32),
        pltpu.SemaphoreType.DMA,
    ],
)
def _sc_spmm2(g3_hbm, src_hbm, dst_hbm, zeros_hbm, out_hbm,
              sidx_v, didx_v, rows_v, acc, sem):
    c = lax.axis_index("c")
    s = lax.axis_index("s")
    pltpu.sync_copy(zeros_hbm, acc.at[pl.ds(s * RPT, RPT)])
    plsc.subcore_barrier()

    def body(i, carry):
        base = (c * 16 + s) * EPT2 + i * C
        pltpu.sync_copy(src_hbm.at[pl.ds(base, C)], sidx_v)
        pltpu.async_copy(g3_hbm.at[sidx_v], rows_v, sem).wait()
        pltpu.sync_copy(dst_hbm.at[pl.ds(base, C)], didx_v)
        pltpu.sync_copy(rows_v, acc.at[didx_v], add=True)
        return carry

    lax.fori_loop(0, EPT2 // C, body, 0)
    plsc.subcore_barrier()
    pltpu.sync_copy(
        acc.at[pl.ds(s * RPT, RPT)], out_hbm.at[pl.ds(c * NP + s * RPT, RPT)]
    )


BM = 512
BN = 1024


def _decoder_body(z_row_ref, z_col_ref, out_ref):
    zi = z_row_ref[...]
    zj = z_col_ref[...]
    logits = jax.lax.dot_general(
        zi, zj, (((1,), (1,)), ((), ())), preferred_element_type=jnp.float32
    )
    out_ref[...] = jax.nn.sigmoid(logits)


def _decoder(z):
    n = z.shape[0]
    grid = (pl.cdiv(n, BM), pl.cdiv(n, BN))
    return pl.pallas_call(
        _decoder_body,
        grid=grid,
        in_specs=[
            pl.BlockSpec((BM, LATENT), lambda i, j: (i, 0)),
            pl.BlockSpec((BN, LATENT), lambda i, j: (j, 0)),
        ],
        out_specs=pl.BlockSpec((BM, BN), lambda i, j: (i, j)),
        out_shape=jax.ShapeDtypeStruct((n, n), jnp.float32),
    )(z, z)


def kernel(x, mask, edge_index, W1, b1, logp, means, logvars, W_mu, b_mu, W_lv, b_lv):
    src = edge_index[0].astype(jnp.int32)
    dst = edge_index[1].astype(jnp.int32)
    padi = jnp.full((EPAD - E,), N, jnp.int32)
    srcp = jnp.concatenate([src, padi])
    dstp = jnp.concatenate([dst, padi])

    ones8 = jnp.ones((C, 8), jnp.float32)
    zeros8 = jnp.zeros((RPT, 8), jnp.float32)
    zerosW = jnp.zeros((RPT, W1COL), jnp.float32)
    zerosH = jnp.zeros((RPT, NHID), jnp.float32)

    idx2 = jnp.concatenate([dstp, srcp])
    deg = _sc_degree(idx2, ones8, zeros8)
    deg_r = deg[:NP, 0] + 1.0
    deg_c = deg[NP:, 0] + 1.0
    ir = lax.rsqrt(deg_r)
    ic = lax.rsqrt(deg_c)

    xpad = jnp.pad(x, ((0, NP - N), (0, 0)))
    Mf = jnp.pad(mask.astype(jnp.float32), ((0, NP - N), (0, 0)))
    xm = xpad - Mf * xpad
    TX0 = xm @ W1
    G1 = ic[:, None] * jnp.concatenate([TX0, Mf], axis=1)
    G2 = (ic * ic)[:, None] * Mf
    G2p = jnp.pad(G2, ((0, 0), (0, W1COL - D_FEAT)))
    gbig = jnp.concatenate([G1, G2p], axis=0)

    src2 = jnp.concatenate([srcp, srcp + NP])
    S = _sc_spmm1(gbig, src2, dstp, zerosW)
    S1 = S[:NP]
    S2 = S[NP:, :D_FEAT]

    full1 = ir[:, None] * (S1 + G1)
    S0 = full1[:, :NHID]
    SM1 = full1[:, NHID:]
    SM2 = (ir * ir)[:, None] * (S2 + G2)

    variances = jnp.exp(logvars)
    Bk = means[:, :, None] * W1[None, :, :]
    Ck = variances[:, :, None] * (W1 * W1)[None, :, :]
    cx = S0[None] + jnp.einsum('nd,kdh->knh', SM1, Bk) + b1
    cc = jnp.einsum('nd,kdh->knh', SM2, Ck)

    std = jnp.sqrt(cc + 1e-8)
    r = cx / std
    cdf = 0.5 * (1.0 + jax.lax.erf(r / jnp.sqrt(2.0)))
    pdf = jnp.exp(-0.5 * r * r) / jnp.sqrt(2.0 * jnp.pi)
    expected_relu = cx * cdf + std * pdf

    U = 1.0 - Mf
    xm2 = xm * xm
    V1 = (1.0 / variances).T
    V2 = (means / variances).T
    V3 = (means * means / variances).T
    quad = xm2 @ V1 - 2.0 * (xm @ V2) + U @ V3
    const_k = 0.5 * D_FEAT * jnp.log(2.0 * jnp.pi) + 0.5 * jnp.sum(logvars, axis=1)
    log_n = (-0.5 * quad - const_k[None, :]).T
    gamma = jax.nn.softmax(logp[:, None] + log_n, axis=0)
    h1 = jnp.sum(gamma[:, :, None] * expected_relu, axis=0)

    H2 = h1 @ jnp.concatenate([W_mu, W_lv], axis=1)
    rowmask = (jnp.arange(NP) < N)[:, None]
    G3 = jnp.where(rowmask, ic[:, None] * H2, 0.0)
    S3cat = _sc_spmm2(G3, srcp, dstp, zerosH)
    S3 = S3cat[:NP] + S3cat[NP:]

    F = ir[:, None] * (S3 + G3)
    mu = F[:N, :LATENT] + b_mu
    logvar = F[:N, LATENT:] + b_lv
    z = mu
    adj_recon = _decoder(z)
    return (adj_recon, z, mu, logvar)
